# baseline (speedup 1.0000x reference)
import functools

import jax
import jax.numpy as jnp
from jax import lax
from jax.experimental import pallas as pl
from jax.experimental.pallas import tpu as pltpu
from jax.experimental.pallas import tpu_sc as plsc

N = 10000
NP = 10240
E = 320000
D = 128
G = 64
NS = 16
NC = 2
C = 128
NCH = 160
EP2 = NS * NCH * C
RPT = NP // NS
ZR = 64
BN = 2048
NB = NP // BN

_mesh = plsc.VectorSubcoreMesh(core_axis_name="c", subcore_axis_name="s",
                               num_cores=NC, num_subcores=NS)


def _deg_body(edges_hbm, out_hbm, idx_b, ones_v, zbuf, acc):
    c = lax.axis_index("c")
    s = lax.axis_index("s")
    zero16 = jnp.zeros((16,), jnp.float32)
    one16 = jnp.ones((16,), jnp.float32)

    @pl.loop(0, ZR)
    def _(r):
        for j in range(D // 16):
            zbuf[r, pl.ds(j * 16, 16)] = zero16

    @pl.loop(0, C)
    def _(r):
        for j in range(D // 16):
            ones_v[r, pl.ds(j * 16, 16)] = one16

    base_r = pl.multiple_of(s * RPT, 8)
    for k in range(RPT // ZR):
        pltpu.sync_copy(zbuf, acc.at[pl.ds(base_r + k * ZR, ZR)])
    plsc.subcore_barrier()

    @pl.loop(0, NCH)
    def _(i):
        pltpu.sync_copy(edges_hbm.at[c, s, i], idx_b)
        pltpu.sync_copy(ones_v, acc.at[idx_b.at[1]], add=True)

    plsc.subcore_barrier()
    pltpu.sync_copy(acc.at[pl.ds(base_r, RPT)], out_hbm.at[c, pl.ds(base_r, RPT)])


def _make_deg(interpret=False):
    return pl.kernel(
        _deg_body,
        out_type=jax.ShapeDtypeStruct((NC, NP, D), jnp.float32),
        mesh=_mesh,
        scratch_types=[
            pltpu.VMEM((2, C), jnp.int32),
            pltpu.VMEM((C, D), jnp.float32),
            pltpu.VMEM((ZR, D), jnp.float32),
            pltpu.VMEM_SHARED((NP, D), jnp.float32),
        ],
        interpret=interpret,
    )


_deg_pass = _make_deg()


def _edge_body(y_hbm, edges_hbm, out_hbm, idx0, idx1, rows0, rows1,
               zbuf, acc, semi0, semi1, semg0, semg1):
    c = lax.axis_index("c")
    s = lax.axis_index("s")
    zero16 = jnp.zeros((16,), jnp.float32)

    @pl.loop(0, ZR)
    def _(r):
        for j in range(D // 16):
            zbuf[r, pl.ds(j * 16, 16)] = zero16

    base_r = pl.multiple_of(s * RPT, 8)
    for k in range(RPT // ZR):
        pltpu.sync_copy(zbuf, acc.at[pl.ds(base_r + k * ZR, ZR)])
    plsc.subcore_barrier()

    def fire_idx(i, idx, semi):
        pltpu.async_copy(edges_hbm.at[c, s, i], idx, semi)

    def fire_gather(i, idx, semi, rows, semg):
        pltpu.make_async_copy(edges_hbm.at[c, s, i], idx, semi).wait()
        pltpu.async_copy(y_hbm.at[idx.at[0]], rows, semg)

    def drain(i, idx, rows, semg):
        pltpu.make_async_copy(y_hbm.at[idx.at[0]], rows, semg).wait()
        pltpu.sync_copy(rows, acc.at[idx.at[1]], add=True)

    fire_idx(0, idx0, semi0)
    fire_idx(1, idx1, semi1)
    fire_gather(0, idx0, semi0, rows0, semg0)

    @pl.loop(0, NCH // 2 - 1)
    def _(k):
        i0 = k * 2
        fire_gather(i0 + 1, idx1, semi1, rows1, semg1)
        drain(i0, idx0, rows0, semg0)
        fire_idx(i0 + 2, idx0, semi0)
        fire_gather(i0 + 2, idx0, semi0, rows0, semg0)
        drain(i0 + 1, idx1, rows1, semg1)
        fire_idx(i0 + 3, idx1, semi1)

    fire_gather(NCH - 1, idx1, semi1, rows1, semg1)
    drain(NCH - 2, idx0, rows0, semg0)
    drain(NCH - 1, idx1, rows1, semg1)

    plsc.subcore_barrier()
    pltpu.sync_copy(acc.at[pl.ds(base_r, RPT)], out_hbm.at[c, pl.ds(base_r, RPT)])


def _make_edge(interpret=False):
    return pl.kernel(
        _edge_body,
        out_type=jax.ShapeDtypeStruct((NC, NP, D), jnp.float32),
        mesh=_mesh,
        scratch_types=[
            pltpu.VMEM((2, C), jnp.int32),
            pltpu.VMEM((2, C), jnp.int32),
            pltpu.VMEM((C, D), jnp.float32),
            pltpu.VMEM((C, D), jnp.float32),
            pltpu.VMEM((ZR, D), jnp.float32),
            pltpu.VMEM_SHARED((NP, D), jnp.float32),
            pltpu.SemaphoreType.DMA,
            pltpu.SemaphoreType.DMA,
            pltpu.SemaphoreType.DMA,
            pltpu.SemaphoreType.DMA,
        ],
        interpret=interpret,
    )


_edge_pass = _make_edge()


def _mm_scale_body(x_ref, w_ref, deg_ref, y_ref):
    dinv = lax.rsqrt(deg_ref[0, :, :1] + 1.0)
    xw = jnp.dot(x_ref[0], w_ref[0], preferred_element_type=jnp.float32, precision=lax.Precision.HIGHEST)
    y_ref[0] = xw * dinv


_mm_scale = pl.pallas_call(
    _mm_scale_body,
    grid=(NC, NB),
    in_specs=[
        pl.BlockSpec((1, BN, D), lambda g, j: (g, j, 0)),
        pl.BlockSpec((1, D, D), lambda g, j: (g, 0, 0)),
        pl.BlockSpec((1, BN, D), lambda g, j: (g, j, 0)),
    ],
    out_specs=pl.BlockSpec((1, BN, D), lambda g, j: (g, j, 0)),
    out_shape=jax.ShapeDtypeStruct((NC, NP, D), jnp.float32),
)


def _epi_mm_body(y_ref, acc_ref, deg_ref, b_ref, w_ref, y2_ref):
    dinv = lax.rsqrt(deg_ref[0, :, :1] + 1.0)
    h = dinv * (y_ref[0] + acc_ref[0]) + b_ref[0]
    h = jnp.where(h >= 0.0, h, 0.01 * h)
    y2_ref[0] = jnp.dot(h, w_ref[0], preferred_element_type=jnp.float32, precision=lax.Precision.HIGHEST) * dinv


_epi_mm = pl.pallas_call(
    _epi_mm_body,
    grid=(NC, NB),
    in_specs=[
        pl.BlockSpec((1, BN, D), lambda g, j: (g, j, 0)),
        pl.BlockSpec((1, BN, D), lambda g, j: (g, j, 0)),
        pl.BlockSpec((1, BN, D), lambda g, j: (g, j, 0)),
        pl.BlockSpec((1, 1, D), lambda g, j: (g, 0, 0)),
        pl.BlockSpec((1, D, D), lambda g, j: (g, 0, 0)),
    ],
    out_specs=pl.BlockSpec((1, BN, D), lambda g, j: (g, j, 0)),
    out_shape=jax.ShapeDtypeStruct((NC, NP, D), jnp.float32),
)


def _pool_body(y_ref, acc_ref, deg_ref, b_ref, batch_ref, sum_ref, cnt_ref):
    j = pl.program_id(1)
    dinv = lax.rsqrt(deg_ref[0, :, :1] + 1.0)
    gblk = dinv * (y_ref[0] + acc_ref[0]) + b_ref[0]
    bt = batch_ref[0, 0, 0]
    oh = (bt[:, None] == lax.broadcasted_iota(jnp.int32, (BN, G), 1)).astype(jnp.float32)
    ps = lax.dot_general(oh, gblk, (((0,), (0,)), ((), ())),
                         preferred_element_type=jnp.float32, precision=lax.Precision.HIGHEST)
    pc = lax.dot_general(oh, jnp.ones((BN, D), jnp.float32), (((0,), (0,)), ((), ())),
                         preferred_element_type=jnp.float32, precision=lax.Precision.HIGHEST)

    @pl.when(j == 0)
    def _():
        sum_ref[0] = ps
        cnt_ref[0] = pc

    @pl.when(j != 0)
    def _():
        sum_ref[0] += ps
        cnt_ref[0] += pc


_pool = pl.pallas_call(
    _pool_body,
    grid=(NC, NB),
    in_specs=[
        pl.BlockSpec((1, BN, D), lambda g, j: (g, j, 0)),
        pl.BlockSpec((1, BN, D), lambda g, j: (g, j, 0)),
        pl.BlockSpec((1, BN, D), lambda g, j: (g, j, 0)),
        pl.BlockSpec((1, 1, D), lambda g, j: (g, 0, 0)),
        pl.BlockSpec((1, 1, 1, BN), lambda g, j: (g, j, 0, 0)),
    ],
    out_specs=[
        pl.BlockSpec((1, G, D), lambda g, j: (g, 0, 0)),
        pl.BlockSpec((1, G, D), lambda g, j: (g, 0, 0)),
    ],
    out_shape=[
        jax.ShapeDtypeStruct((NC, G, D), jnp.float32),
        jax.ShapeDtypeStruct((NC, G, D), jnp.float32),
    ],
)


def _ln(x, g, b):
    m = jnp.mean(x, axis=-1, keepdims=True)
    v = jnp.mean((x - m) ** 2, axis=-1, keepdims=True)
    return (x - m) / jnp.sqrt(v + 1e-5) * g + b


def _attn_body(sum_ref, cnt_ref, wq, bq, wk, bk, wv, bv, wo, bo, g1, be1,
               wf1, bf1, wf2, bf2, g2, be2, wl1, bl1, wl2r, bl2, out_ref):
    h = sum_ref[...] / jnp.maximum(cnt_ref[...], 1.0)
    hA = h[0]
    hB = h[1]

    def cross(xq, xkv):
        Q = jnp.dot(xq, wq[...], preferred_element_type=jnp.float32, precision=lax.Precision.HIGHEST) + bq[...]
        K = jnp.dot(xkv, wk[...], preferred_element_type=jnp.float32, precision=lax.Precision.HIGHEST) + bk[...]
        V = jnp.dot(xkv, wv[...], preferred_element_type=jnp.float32, precision=lax.Precision.HIGHEST) + bv[...]
        dh = D // 4
        outs = []
        for hh in range(4):
            Qh = Q[:, hh * dh:(hh + 1) * dh]
            Kh = K[:, hh * dh:(hh + 1) * dh]
            Vh = V[:, hh * dh:(hh + 1) * dh]
            a = lax.dot_general(Qh, Kh, (((1,), (1,)), ((), ())),
                                preferred_element_type=jnp.float32, precision=lax.Precision.HIGHEST) * (dh ** -0.5)
            a = a - jnp.max(a, axis=1, keepdims=True)
            ea = jnp.exp(a)
            sm = ea / jnp.sum(ea, axis=1, keepdims=True)
            outs.append(jnp.dot(sm, Vh, preferred_element_type=jnp.float32, precision=lax.Precision.HIGHEST))
        o = jnp.concatenate(outs, axis=1)
        o = jnp.dot(o, wo[...], preferred_element_type=jnp.float32, precision=lax.Precision.HIGHEST) + bo[...]
        x = _ln(xq + o, g1[...], be1[...])
        f = jnp.dot(x, wf1[...], preferred_element_type=jnp.float32, precision=lax.Precision.HIGHEST) + bf1[...]
        f = jnp.where(f >= 0.0, f, 0.01 * f)
        f = jnp.dot(f, wf2[...], preferred_element_type=jnp.float32, precision=lax.Precision.HIGHEST) + bf2[...]
        return _ln(x + f, g2[...], be2[...])

    cat = jnp.concatenate([cross(hA, hB), cross(hB, hA)], axis=1)
    l1 = jnp.dot(cat, wl1[...], preferred_element_type=jnp.float32, precision=lax.Precision.HIGHEST) + bl1[...]
    l1 = jnp.maximum(l1, 0.0)
    out_ref[...] = jnp.sum(l1 * wl2r[...], axis=1, keepdims=True) + bl2[0, 0]


_attn = pl.pallas_call(
    _attn_body,
    out_shape=jax.ShapeDtypeStruct((G, 1), jnp.float32),
)


def kernel(x1, edge_index1, batch1, x2, edge_index2, batch2, wA1, bA1, wA2, bA2,
           wB1, bB1, wB2, bB2, wq, bq, wk, bk, wv, bv, wo, bo, g1, be1, wf1, bf1,
           wf2, bf2, g2, be2, wl1, bl1, wl2, bl2):
    xs = jnp.pad(jnp.stack([x1, x2]), ((0, 0), (0, NP - N), (0, 0)))
    pad_e = jnp.full((EP2 - E,), N, jnp.int32)
    dst1f = jnp.concatenate([edge_index1[1], pad_e])
    dst2f = jnp.concatenate([edge_index2[1], pad_e])
    src1 = jnp.concatenate([edge_index1[0], pad_e]).reshape(NS, NCH, 1, C)
    dst1 = dst1f.reshape(NS, NCH, 1, C)
    src2 = (jnp.concatenate([edge_index2[0], pad_e]) + NP).reshape(NS, NCH, 1, C)
    dst2 = dst2f.reshape(NS, NCH, 1, C)
    edges = jnp.stack([jnp.concatenate([src1, dst1], axis=2),
                       jnp.concatenate([src2, dst2], axis=2)])
    batch_r = jnp.pad(jnp.stack([batch1, batch2]), ((0, 0), (0, NP - N)),
                      constant_values=G).reshape(NC, NB, 1, BN)
    w1 = jnp.stack([wA1, wB1])
    b1 = jnp.stack([bA1, bB1]).reshape(NC, 1, D)
    w2 = jnp.stack([wA2, wB2])
    b2 = jnp.stack([bA2, bB2]).reshape(NC, 1, D)

    deg16 = _deg_pass(edges)
    y = _mm_scale(xs, w1, deg16)
    acc = _edge_pass(y.reshape(NC * NP, D), edges)
    y2 = _epi_mm(y, acc, deg16, b1, w2)
    acc2 = _edge_pass(y2.reshape(NC * NP, D), edges)
    sums, cnts = _pool(y2, acc2, deg16, b2, batch_r)
    return _attn(sums, cnts, wq, bq.reshape(1, D), wk, bk.reshape(1, D),
                 wv, bv.reshape(1, D), wo, bo.reshape(1, D),
                 g1.reshape(1, D), be1.reshape(1, D), wf1, bf1.reshape(1, 2 * D),
                 wf2, bf2.reshape(1, D), g2.reshape(1, D), be2.reshape(1, D),
                 wl1, bl1.reshape(1, D), wl2.reshape(1, D), bl2.reshape(1, 1))


# speedup vs baseline: 2.0274x; 2.0274x over previous
import functools

import jax
import jax.numpy as jnp
from jax import lax
from jax.experimental import pallas as pl
from jax.experimental.pallas import tpu as pltpu
from jax.experimental.pallas import tpu_sc as plsc

N = 10000
NP = 10240
E = 320000
D = 128
G = 64
NS = 16
NC = 2
C = 128
NCH = 160
EP2 = NS * NCH * C
RPT = NP // NS
ZR = 64
BN = 2048
NB = NP // BN

_mesh = plsc.VectorSubcoreMesh(core_axis_name="c", subcore_axis_name="s",
                               num_cores=NC, num_subcores=NS)


def _deg_body(edges_hbm, out_hbm, idx_b, ones_v, zbuf, acc):
    c = lax.axis_index("c")
    s = lax.axis_index("s")
    zero16 = jnp.zeros((16,), jnp.float32)
    one16 = jnp.ones((16,), jnp.float32)

    @pl.loop(0, ZR)
    def _(r):
        for j in range(D // 16):
            zbuf[r, pl.ds(j * 16, 16)] = zero16

    @pl.loop(0, C)
    def _(r):
        for j in range(D // 16):
            ones_v[r, pl.ds(j * 16, 16)] = one16

    base_r = pl.multiple_of(s * RPT, 8)
    for k in range(RPT // ZR):
        pltpu.sync_copy(zbuf, acc.at[pl.ds(base_r + k * ZR, ZR)])
    plsc.subcore_barrier()

    @pl.loop(0, NCH)
    def _(i):
        pltpu.sync_copy(edges_hbm.at[c, s, i], idx_b)
        pltpu.sync_copy(ones_v, acc.at[idx_b.at[1]], add=True)

    plsc.subcore_barrier()
    pltpu.sync_copy(acc.at[pl.ds(base_r, RPT)], out_hbm.at[c, pl.ds(base_r, RPT)])


def _make_deg(interpret=False):
    return pl.kernel(
        _deg_body,
        out_type=jax.ShapeDtypeStruct((NC, NP, D), jnp.float32),
        mesh=_mesh,
        scratch_types=[
            pltpu.VMEM((2, C), jnp.int32),
            pltpu.VMEM((C, D), jnp.float32),
            pltpu.VMEM((ZR, D), jnp.float32),
            pltpu.VMEM_SHARED((NP, D), jnp.float32),
        ],
        interpret=interpret,
    )


_deg_pass = _make_deg()


def _edge_body(y_hbm, edges_hbm, out_hbm, idx0, idx1, rows0, rows1,
               zbuf, acc, semi0, semi1, semg0, semg1):
    c = lax.axis_index("c")
    s = lax.axis_index("s")
    zero16 = jnp.zeros((16,), jnp.float32)

    @pl.loop(0, ZR)
    def _(r):
        for j in range(D // 16):
            zbuf[r, pl.ds(j * 16, 16)] = zero16

    base_r = pl.multiple_of(s * RPT, 8)
    for k in range(RPT // ZR):
        pltpu.sync_copy(zbuf, acc.at[pl.ds(base_r + k * ZR, ZR)])
    plsc.subcore_barrier()

    def fire_idx(i, idx, semi):
        pltpu.async_copy(edges_hbm.at[c, s, i], idx, semi)

    def fire_gather(i, idx, semi, rows, semg):
        pltpu.make_async_copy(edges_hbm.at[c, s, i], idx, semi).wait()
        pltpu.async_copy(y_hbm.at[idx.at[0]], rows, semg)

    def drain(i, idx, rows, semg):
        pltpu.make_async_copy(y_hbm.at[idx.at[0]], rows, semg).wait()
        pltpu.sync_copy(rows, acc.at[idx.at[1]], add=True)

    fire_idx(0, idx0, semi0)
    fire_idx(1, idx1, semi1)
    fire_gather(0, idx0, semi0, rows0, semg0)

    @pl.loop(0, NCH // 2 - 1)
    def _(k):
        i0 = k * 2
        fire_gather(i0 + 1, idx1, semi1, rows1, semg1)
        drain(i0, idx0, rows0, semg0)
        fire_idx(i0 + 2, idx0, semi0)
        fire_gather(i0 + 2, idx0, semi0, rows0, semg0)
        drain(i0 + 1, idx1, rows1, semg1)
        fire_idx(i0 + 3, idx1, semi1)

    fire_gather(NCH - 1, idx1, semi1, rows1, semg1)
    drain(NCH - 2, idx0, rows0, semg0)
    drain(NCH - 1, idx1, rows1, semg1)

    plsc.subcore_barrier()
    pltpu.sync_copy(acc.at[pl.ds(base_r, RPT)], out_hbm.at[c, pl.ds(base_r, RPT)])


def _make_edge(interpret=False):
    return pl.kernel(
        _edge_body,
        out_type=jax.ShapeDtypeStruct((NC, NP, D), jnp.float32),
        mesh=_mesh,
        scratch_types=[
            pltpu.VMEM((2, C), jnp.int32),
            pltpu.VMEM((2, C), jnp.int32),
            pltpu.VMEM((C, D), jnp.float32),
            pltpu.VMEM((C, D), jnp.float32),
            pltpu.VMEM((ZR, D), jnp.float32),
            pltpu.VMEM_SHARED((NP, D), jnp.float32),
            pltpu.SemaphoreType.DMA,
            pltpu.SemaphoreType.DMA,
            pltpu.SemaphoreType.DMA,
            pltpu.SemaphoreType.DMA,
        ],
        interpret=interpret,
    )


_edge_pass = _make_edge()


def _mm_scale_body(x_ref, w_ref, deg_ref, y_ref):
    dinv = lax.rsqrt(deg_ref[0, :, :1] + 1.0)
    xw = jnp.dot(x_ref[0], w_ref[0], preferred_element_type=jnp.float32, precision=lax.Precision.HIGHEST)
    y_ref[0] = xw * dinv


_mm_scale = pl.pallas_call(
    _mm_scale_body,
    grid=(NC, NB),
    in_specs=[
        pl.BlockSpec((1, BN, D), lambda g, j: (g, j, 0)),
        pl.BlockSpec((1, D, D), lambda g, j: (g, 0, 0)),
        pl.BlockSpec((1, BN, D), lambda g, j: (g, j, 0)),
    ],
    out_specs=pl.BlockSpec((1, BN, D), lambda g, j: (g, j, 0)),
    out_shape=jax.ShapeDtypeStruct((NC, NP, D), jnp.float32),
)


def _epi_mm_body(y_ref, acc_ref, deg_ref, b_ref, w_ref, y2_ref):
    dinv = lax.rsqrt(deg_ref[0, :, :1] + 1.0)
    h = dinv * (y_ref[0] + acc_ref[0]) + b_ref[0]
    h = jnp.where(h >= 0.0, h, 0.01 * h)
    y2_ref[0] = jnp.dot(h, w_ref[0], preferred_element_type=jnp.float32, precision=lax.Precision.HIGHEST) * dinv


_epi_mm = pl.pallas_call(
    _epi_mm_body,
    grid=(NC, NB),
    in_specs=[
        pl.BlockSpec((1, BN, D), lambda g, j: (g, j, 0)),
        pl.BlockSpec((1, BN, D), lambda g, j: (g, j, 0)),
        pl.BlockSpec((1, BN, D), lambda g, j: (g, j, 0)),
        pl.BlockSpec((1, 1, D), lambda g, j: (g, 0, 0)),
        pl.BlockSpec((1, D, D), lambda g, j: (g, 0, 0)),
    ],
    out_specs=pl.BlockSpec((1, BN, D), lambda g, j: (g, j, 0)),
    out_shape=jax.ShapeDtypeStruct((NC, NP, D), jnp.float32),
)


def _pool_body(y_ref, acc_ref, deg_ref, b_ref, batch_ref, sum_ref, cnt_ref):
    j = pl.program_id(1)
    dinv = lax.rsqrt(deg_ref[0, :, :1] + 1.0)
    gblk = dinv * (y_ref[0] + acc_ref[0]) + b_ref[0]
    bt = batch_ref[0, 0, 0]
    oh = (bt[:, None] == lax.broadcasted_iota(jnp.int32, (BN, G), 1)).astype(jnp.float32)
    ps = lax.dot_general(oh, gblk, (((0,), (0,)), ((), ())),
                         preferred_element_type=jnp.float32, precision=lax.Precision.HIGHEST)
    pc = lax.dot_general(oh, jnp.ones((BN, D), jnp.float32), (((0,), (0,)), ((), ())),
                         preferred_element_type=jnp.float32, precision=lax.Precision.HIGHEST)

    @pl.when(j == 0)
    def _():
        sum_ref[0] = ps
        cnt_ref[0] = pc

    @pl.when(j != 0)
    def _():
        sum_ref[0] += ps
        cnt_ref[0] += pc


_pool = pl.pallas_call(
    _pool_body,
    grid=(NC, NB),
    in_specs=[
        pl.BlockSpec((1, BN, D), lambda g, j: (g, j, 0)),
        pl.BlockSpec((1, BN, D), lambda g, j: (g, j, 0)),
        pl.BlockSpec((1, BN, D), lambda g, j: (g, j, 0)),
        pl.BlockSpec((1, 1, D), lambda g, j: (g, 0, 0)),
        pl.BlockSpec((1, 1, 1, BN), lambda g, j: (g, j, 0, 0)),
    ],
    out_specs=[
        pl.BlockSpec((1, G, D), lambda g, j: (g, 0, 0)),
        pl.BlockSpec((1, G, D), lambda g, j: (g, 0, 0)),
    ],
    out_shape=[
        jax.ShapeDtypeStruct((NC, G, D), jnp.float32),
        jax.ShapeDtypeStruct((NC, G, D), jnp.float32),
    ],
)


def _ln(x, g, b):
    m = jnp.mean(x, axis=-1, keepdims=True)
    v = jnp.mean((x - m) ** 2, axis=-1, keepdims=True)
    return (x - m) / jnp.sqrt(v + 1e-5) * g + b


def _attn_body(sum_ref, cnt_ref, wq, bq, wk, bk, wv, bv, wo, bo, g1, be1,
               wf1, bf1, wf2, bf2, g2, be2, wl1, bl1, wl2r, bl2, out_ref):
    h = sum_ref[...] / jnp.maximum(cnt_ref[...], 1.0)
    hA = h[0]
    hB = h[1]

    def cross(xq, xkv):
        Q = jnp.dot(xq, wq[...], preferred_element_type=jnp.float32, precision=lax.Precision.HIGHEST) + bq[...]
        K = jnp.dot(xkv, wk[...], preferred_element_type=jnp.float32, precision=lax.Precision.HIGHEST) + bk[...]
        V = jnp.dot(xkv, wv[...], preferred_element_type=jnp.float32, precision=lax.Precision.HIGHEST) + bv[...]
        dh = D // 4
        outs = []
        for hh in range(4):
            Qh = Q[:, hh * dh:(hh + 1) * dh]
            Kh = K[:, hh * dh:(hh + 1) * dh]
            Vh = V[:, hh * dh:(hh + 1) * dh]
            a = lax.dot_general(Qh, Kh, (((1,), (1,)), ((), ())),
                                preferred_element_type=jnp.float32, precision=lax.Precision.HIGHEST) * (dh ** -0.5)
            a = a - jnp.max(a, axis=1, keepdims=True)
            ea = jnp.exp(a)
            sm = ea / jnp.sum(ea, axis=1, keepdims=True)
            outs.append(jnp.dot(sm, Vh, preferred_element_type=jnp.float32, precision=lax.Precision.HIGHEST))
        o = jnp.concatenate(outs, axis=1)
        o = jnp.dot(o, wo[...], preferred_element_type=jnp.float32, precision=lax.Precision.HIGHEST) + bo[...]
        x = _ln(xq + o, g1[...], be1[...])
        f = jnp.dot(x, wf1[...], preferred_element_type=jnp.float32, precision=lax.Precision.HIGHEST) + bf1[...]
        f = jnp.where(f >= 0.0, f, 0.01 * f)
        f = jnp.dot(f, wf2[...], preferred_element_type=jnp.float32, precision=lax.Precision.HIGHEST) + bf2[...]
        return _ln(x + f, g2[...], be2[...])

    cat = jnp.concatenate([cross(hA, hB), cross(hB, hA)], axis=1)
    l1 = jnp.dot(cat, wl1[...], preferred_element_type=jnp.float32, precision=lax.Precision.HIGHEST) + bl1[...]
    l1 = jnp.maximum(l1, 0.0)
    out_ref[...] = jnp.sum(l1 * wl2r[...], axis=1, keepdims=True) + bl2[0, 0]


_attn = pl.pallas_call(
    _attn_body,
    out_shape=jax.ShapeDtypeStruct((G, 1), jnp.float32),
)


def kernel(x1, edge_index1, batch1, x2, edge_index2, batch2, wA1, bA1, wA2, bA2,
           wB1, bB1, wB2, bB2, wq, bq, wk, bk, wv, bv, wo, bo, g1, be1, wf1, bf1,
           wf2, bf2, g2, be2, wl1, bl1, wl2, bl2):
    xs = jnp.pad(jnp.stack([x1, x2]), ((0, 0), (0, NP - N), (0, 0)))
    pad_e = N + (jnp.arange(EP2 - E, dtype=jnp.int32) % (NP - N))
    dst1f = jnp.concatenate([edge_index1[1], pad_e])
    dst2f = jnp.concatenate([edge_index2[1], pad_e])
    src1 = jnp.concatenate([edge_index1[0], pad_e]).reshape(NS, NCH, 1, C)
    dst1 = dst1f.reshape(NS, NCH, 1, C)
    src2 = (jnp.concatenate([edge_index2[0], pad_e]) + NP).reshape(NS, NCH, 1, C)
    dst2 = dst2f.reshape(NS, NCH, 1, C)
    edges = jnp.stack([jnp.concatenate([src1, dst1], axis=2),
                       jnp.concatenate([src2, dst2], axis=2)])
    batch_r = jnp.pad(jnp.stack([batch1, batch2]), ((0, 0), (0, NP - N)),
                      constant_values=G).reshape(NC, NB, 1, BN)
    w1 = jnp.stack([wA1, wB1])
    b1 = jnp.stack([bA1, bB1]).reshape(NC, 1, D)
    w2 = jnp.stack([wA2, wB2])
    b2 = jnp.stack([bA2, bB2]).reshape(NC, 1, D)

    deg16 = _deg_pass(edges)
    y = _mm_scale(xs, w1, deg16)
    acc = _edge_pass(y.reshape(NC * NP, D), edges)
    y2 = _epi_mm(y, acc, deg16, b1, w2)
    acc2 = _edge_pass(y2.reshape(NC * NP, D), edges)
    sums, cnts = _pool(y2, acc2, deg16, b2, batch_r)
    return _attn(sums, cnts, wq, bq.reshape(1, D), wk, bk.reshape(1, D),
                 wv, bv.reshape(1, D), wo, bo.reshape(1, D),
                 g1.reshape(1, D), be1.reshape(1, D), wf1, bf1.reshape(1, 2 * D),
                 wf2, bf2.reshape(1, D), g2.reshape(1, D), be2.reshape(1, D),
                 wl1, bl1.reshape(1, D), wl2.reshape(1, D), bl2.reshape(1, 1))
